# baseline (speedup 1.0000x reference)
import jax
import jax.numpy as jnp
from jax.experimental import pallas as pl
from jax.experimental.pallas import tpu as pltpu


_TM = 512


def _interp_tile(s0, s1, s2, s3, x_ref, o_ref):
    kq = s0.shape[1]
    acc = jnp.dot(s0[...], x_ref[0 * kq:1 * kq, :],
                  preferred_element_type=jnp.float32)
    acc += jnp.dot(s1[...], x_ref[1 * kq:2 * kq, :],
                   preferred_element_type=jnp.float32)
    acc += jnp.dot(s2[...], x_ref[2 * kq:3 * kq, :],
                   preferred_element_type=jnp.float32)
    acc += jnp.dot(s3[...], x_ref[3 * kq:4 * kq, :],
                   preferred_element_type=jnp.float32)
    o_ref[...] = acc


def kernel(x_coarse, interp_matrix):
    m, k = interp_matrix.shape
    n = x_coarse.shape[1]
    kq = k // 4
    return pl.pallas_call(
        _interp_tile,
        grid=(m // _TM,),
        in_specs=[
            pl.BlockSpec((_TM, kq), lambda i: (i, 0)),
            pl.BlockSpec((_TM, kq), lambda i: (i, 1)),
            pl.BlockSpec((_TM, kq), lambda i: (i, 2)),
            pl.BlockSpec((_TM, kq), lambda i: (i, 3)),
            pl.BlockSpec(memory_space=pltpu.MemorySpace.VMEM),
        ],
        out_specs=pl.BlockSpec((_TM, n), lambda i: (i, 0)),
        out_shape=jax.ShapeDtypeStruct((m, n), jnp.float32),
    )(interp_matrix, interp_matrix, interp_matrix, interp_matrix, x_coarse)
